# baseline (speedup 1.0000x reference)
import functools

import jax
import jax.numpy as jnp
import numpy as np
from jax import lax
from jax.experimental import pallas as pl
from jax.experimental.pallas import tpu as pltpu
from jax.experimental.pallas import tpu_sc as plsc

N = 10000
E = 320000
D_IN = 128
D_E = 16
H = 256
HH = 128
G = 64

F32 = jnp.float32
BF16 = jnp.bfloat16


def _to_bf16_rtne(x):
    b = jax.lax.bitcast_convert_type(x, jnp.uint32)
    b = b + jnp.uint32(0x7FFF) + ((b >> jnp.uint32(16)) & jnp.uint32(1))
    b = b & jnp.uint32(0xFFFF0000)
    return jax.lax.bitcast_convert_type(b, F32).astype(BF16)

_P128 = np.zeros(128, np.int32)
for _g in range(4):
    for _i in range(16):
        _P128[32 * _g + 2 * _i] = 32 * _g + _i
        _P128[32 * _g + 2 * _i + 1] = 32 * _g + 16 + _i
_P256 = np.concatenate([_P128, 128 + _P128])


def _node_enc_body(xb, wb, bb, ob):
    ob_val = (
        jnp.dot(xb[...], wb[...], preferred_element_type=F32)
        + bb[...].reshape(1, HH)
    )
    ob[...] = _to_bf16_rtne(ob_val)


def _node_encode(x, node_W, node_b2):
    nt = 5
    bm = N // nt
    return pl.pallas_call(
        _node_enc_body,
        grid=(2, nt),
        in_specs=[
            pl.BlockSpec((bm, D_IN), lambda c, i: (i, 0)),
            pl.BlockSpec((D_IN, HH), lambda c, i: (0, c)),
            pl.BlockSpec((1, 1, HH), lambda c, i: (c, 0, 0)),
        ],
        out_specs=pl.BlockSpec((bm, HH), lambda c, i: (c * nt + i, 0)),
        out_shape=jax.ShapeDtypeStruct((2 * N, HH), BF16),
    )(x, node_W, node_b2)


def _edge_proj_body(ab, wb, bb, ob):
    ob[...] = (
        jnp.dot(ab[...], wb[...], preferred_element_type=F32)
        + bb[...].reshape(1, HH)
    ).astype(BF16)


def _edge_proj(edge_attr, eW, eb2):
    nt = 80
    bm = E // nt
    return pl.pallas_call(
        _edge_proj_body,
        grid=(2, nt),
        in_specs=[
            pl.BlockSpec((bm, D_E), lambda c, i: (i, 0)),
            pl.BlockSpec((D_E, HH), lambda c, i: (0, c)),
            pl.BlockSpec((1, 1, HH), lambda c, i: (c, 0, 0)),
        ],
        out_specs=pl.BlockSpec((bm, HH), lambda c, i: (c * nt + i, 0)),
        out_shape=jax.ShapeDtypeStruct((2 * E, HH), BF16),
    )(edge_attr, eW, eb2)


def _mlp_body(h0, h1, a0, a1, w1p, w1, b1, w2, b2, ob):
    hz = jnp.concatenate([h0[...].astype(F32), h1[...].astype(F32)], axis=1)
    az = jnp.concatenate([a0[...], a1[...]], axis=1)
    t = jnp.maximum(
        jnp.dot(hz, w1p[...], preferred_element_type=F32)
        + jnp.dot(az, w1[...], preferred_element_type=F32)
        + b1[...], 0.0)
    u = jnp.dot(t, w2[...], preferred_element_type=F32) + b2[...].reshape(1, HH)
    ob[...] = _to_bf16_rtne(jnp.maximum(u, 0.0))


def _mlp(h_flat, aggr, W1p, W1, b1r, W2, b2r):
    nt = 5
    bm = N // nt
    return pl.pallas_call(
        _mlp_body,
        grid=(2, nt),
        in_specs=[
            pl.BlockSpec((bm, HH), lambda c, i: (i, 0)),
            pl.BlockSpec((bm, HH), lambda c, i: (nt + i, 0)),
            pl.BlockSpec((bm, HH), lambda c, i: (i, 0)),
            pl.BlockSpec((bm, HH), lambda c, i: (nt + i, 0)),
            pl.BlockSpec((H, H), lambda c, i: (0, 0)),
            pl.BlockSpec((H, H), lambda c, i: (0, 0)),
            pl.BlockSpec((1, H), lambda c, i: (0, 0)),
            pl.BlockSpec((H, HH), lambda c, i: (0, c)),
            pl.BlockSpec((1, 1, HH), lambda c, i: (c, 0, 0)),
        ],
        out_specs=pl.BlockSpec((bm, HH), lambda c, i: (c * nt + i, 0)),
        out_shape=jax.ShapeDtypeStruct((2 * N, HH), BF16),
    )(h_flat, h_flat, aggr, aggr, W1p, W1, b1r, W2, b2r)


def _pool_body(h0, h1, bt, w0, w1, bb, ob):
    i = pl.program_id(0)
    y = (
        jnp.dot(h0[...].astype(F32), w0[...], preferred_element_type=F32)
        + jnp.dot(h1[...].astype(F32), w1[...], preferred_element_type=F32)
    )
    bm = y.shape[0]
    oh = (
        lax.broadcasted_iota(jnp.int32, (G, bm), 0)
        == jnp.broadcast_to(bt[...].reshape(1, bm), (G, bm))
    ).astype(F32)
    part = jnp.dot(oh, y, preferred_element_type=F32)

    @pl.when(i == 0)
    def _init():
        ob[...] = jnp.broadcast_to(bb[...], ob.shape)

    ob[...] += part


def _pool_project(h_flat, batch2d, oW0, oW1, ob11):
    nt = 5
    bm = N // nt
    res = pl.pallas_call(
        _pool_body,
        grid=(nt,),
        in_specs=[
            pl.BlockSpec((bm, HH), lambda i: (i, 0)),
            pl.BlockSpec((bm, HH), lambda i: (nt + i, 0)),
            pl.BlockSpec((1, 1, bm), lambda i: (i, 0, 0)),
            pl.BlockSpec((HH, 1), lambda i: (0, 0)),
            pl.BlockSpec((HH, 1), lambda i: (0, 0)),
            pl.BlockSpec((1, 1), lambda i: (0, 0)),
        ],
        out_specs=pl.BlockSpec((G, 1), lambda i: (0, 0)),
        out_shape=jax.ShapeDtypeStruct((G, 1), F32),
    )(h_flat, h_flat, batch2d, oW0, oW1, ob11)
    return res.reshape(G)


CHM = 128
NCM = 156
EPS = NCM * CHM
TAIL = 32
TBASE = 16 * EPS
RPT = 624


def _sc_msg_body(h_hbm, e_hbm, src3_hbm, dst3_hbm, srct_hbm, dstt_hbm, out_hbm,
                 sI0, sI1, dI0, dI1, stx, dtx, hbuf0, hbuf1, ebuf0, ebuf1,
                 msg, acc,
                 semE0, semE1, semS0, semS1, semD0, semD1, semG0, semG1):
    c = lax.axis_index("c")
    s = lax.axis_index("s")
    cN = c * N
    cE = c * E
    sI = (sI0, sI1)
    dI = (dI0, dI1)
    hbuf = (hbuf0, hbuf1)
    ebuf = (ebuf0, ebuf1)
    semE = (semE0, semE1)
    semS = (semS0, semS1)
    semD = (semD0, semD1)
    semG = (semG0, semG1)
    MAXROW = 16 * NCM - 1

    pltpu.sync_copy(srct_hbm.at[s], stx)
    pltpu.sync_copy(dstt_hbm.at[s], dtx)
    for kk in range(TAIL // 16):
        stx[pl.ds(kk * 16, 16)] = stx[pl.ds(kk * 16, 16)] + cN

    zero16 = jnp.zeros((16,), F32)
    zero32 = jnp.zeros((32,), BF16)

    def _zrow(r, carry):
        for kk in range(8):
            msg[r, pl.ds(kk * 16, 16)] = zero16
        return carry

    lax.fori_loop(0, CHM, _zrow, 0)

    arow = s * RPT
    for t in range(RPT // CHM):
        pltpu.sync_copy(msg, acc.at[pl.ds(arow + t * CHM, CHM)])
    rem = RPT - (RPT // CHM) * CHM
    if rem:
        pltpu.sync_copy(msg.at[pl.ds(0, rem)],
                        acc.at[pl.ds(arow + (RPT // CHM) * CHM, rem)])

    @pl.when(s == 15)
    def _ztail():
        pltpu.sync_copy(msg.at[pl.ds(0, N - 16 * RPT)],
                        acc.at[pl.ds(16 * RPT, N - 16 * RPT)])

    plsc.subcore_barrier()

    ebase = cE + s * EPS

    def _prefetch(j, p):
        row = jnp.minimum(s * NCM + j, MAXROW)
        pltpu.async_copy(e_hbm.at[pl.ds(ebase + j * CHM, CHM)], ebuf[p], semE[p])
        pltpu.async_copy(src3_hbm.at[row], sI[p], semS[p])
        pltpu.async_copy(dst3_hbm.at[row], dI[p], semD[p])

    def _start_gather(p):
        pltpu.make_async_copy(src3_hbm.at[0], sI[p], semS[p]).wait()
        for kk in range(8):
            sI[p][pl.ds(kk * 16, 16)] = sI[p][pl.ds(kk * 16, 16)] + cN
        pltpu.async_copy(h_hbm.at[sI[p]], hbuf[p], semG[p])

    def _msg_compute(hb, eb, nrow):
        def _rrow(r, cr):
            for kk in range(4):
                hv = plsc.bitcast(hb[r, pl.ds(kk * 16, 16)], BF16)
                ev = plsc.bitcast(eb[r, pl.ds(kk * 16, 16)], BF16)
                ha_, hb_ = plsc.unpack(hv, format=plsc.PackFormat.INTERLEAVED)
                ea_, eb_ = plsc.unpack(ev, format=plsc.PackFormat.INTERLEAVED)
                msg[r, pl.ds(kk * 32, 16)] = jnp.maximum(ha_ + ea_, zero16)
                msg[r, pl.ds(kk * 32 + 16, 16)] = jnp.maximum(hb_ + eb_, zero16)
            return cr
        lax.fori_loop(0, nrow, _rrow, 0)

    def _finish(p):
        pltpu.make_async_copy(h_hbm.at[sI[p]], hbuf[p], semG[p]).wait()
        pltpu.make_async_copy(e_hbm.at[pl.ds(0, CHM)], ebuf[p], semE[p]).wait()
        _msg_compute(hbuf[p], ebuf[p], CHM)
        pltpu.make_async_copy(dst3_hbm.at[0], dI[p], semD[p]).wait()
        pltpu.sync_copy(msg, acc.at[dI[p]], add=True)

    _prefetch(0, 0)
    _prefetch(1, 1)
    _start_gather(0)

    def _body(i, carry):
        j0 = 2 * i
        _start_gather(1)
        _finish(0)
        _prefetch(j0 + 2, 0)
        _finish(1)
        _prefetch(j0 + 3, 1)

        @pl.when(i < NCM // 2 - 1)
        def _more():
            _start_gather(0)
        return carry

    lax.fori_loop(0, NCM // 2, _body, 0)

    for p in range(2):
        pltpu.make_async_copy(e_hbm.at[pl.ds(0, CHM)], ebuf[p], semE[p]).wait()
        pltpu.make_async_copy(src3_hbm.at[0], sI[p], semS[p]).wait()
        pltpu.make_async_copy(dst3_hbm.at[0], dI[p], semD[p]).wait()

    pltpu.sync_copy(e_hbm.at[pl.ds(cE + TBASE + s * TAIL, TAIL)],
                    ebuf0.at[pl.ds(0, TAIL)])
    pltpu.sync_copy(h_hbm.at[stx], hbuf0.at[pl.ds(0, TAIL)])
    _msg_compute(hbuf0, ebuf0, TAIL)
    pltpu.sync_copy(msg.at[pl.ds(0, TAIL)], acc.at[dtx], add=True)

    plsc.subcore_barrier()
    pltpu.sync_copy(acc.at[pl.ds(arow, RPT)],
                    out_hbm.at[pl.ds(cN + arow, RPT)])

    @pl.when(s == 15)
    def _wtail():
        pltpu.sync_copy(acc.at[pl.ds(16 * RPT, N - 16 * RPT)],
                        out_hbm.at[pl.ds(cN + 16 * RPT, N - 16 * RPT)])


def _sc_message(h_flat, e_flat, src3, dst3, srct, dstt):
    mesh = plsc.VectorSubcoreMesh(core_axis_name="c", subcore_axis_name="s")
    k = functools.partial(
        pl.kernel,
        out_type=jax.ShapeDtypeStruct((2 * N, HH), F32),
        mesh=mesh,
        compiler_params=pltpu.CompilerParams(use_tc_tiling_on_sc=False, needs_layout_passes=False),
        scratch_types=[
            pltpu.VMEM((CHM,), jnp.int32),
            pltpu.VMEM((CHM,), jnp.int32),
            pltpu.VMEM((CHM,), jnp.int32),
            pltpu.VMEM((CHM,), jnp.int32),
            pltpu.VMEM((TAIL,), jnp.int32),
            pltpu.VMEM((TAIL,), jnp.int32),
            pltpu.VMEM((CHM, HH // 2), F32),
            pltpu.VMEM((CHM, HH // 2), F32),
            pltpu.VMEM((CHM, HH // 2), F32),
            pltpu.VMEM((CHM, HH // 2), F32),
            pltpu.VMEM((CHM, HH), F32),
            pltpu.VMEM_SHARED((N, HH), F32),
            pltpu.SemaphoreType.DMA,
            pltpu.SemaphoreType.DMA,
            pltpu.SemaphoreType.DMA,
            pltpu.SemaphoreType.DMA,
            pltpu.SemaphoreType.DMA,
            pltpu.SemaphoreType.DMA,
            pltpu.SemaphoreType.DMA,
            pltpu.SemaphoreType.DMA,
        ],
    )(_sc_msg_body)
    return k(h_flat, e_flat, src3, dst3, srct, dstt)


def kernel(x, edge_index, edge_attr, batch, node_W, node_b,
           edge_W0, edge_b0, W1_0, b1_0, W2_0, b2_0,
           edge_W1, edge_b1, W1_1, b1_1, W2_1, b2_1,
           edge_W2, edge_b2, W1_2, b1_2, W2_2, b2_2,
           out_W, out_b):
    src = edge_index[0]
    dst = edge_index[1]
    srcm = src[:TBASE].reshape(16 * NCM, CHM)
    dstm = dst[:TBASE].reshape(16 * NCM, CHM)
    srct = src[TBASE:].reshape(16, TAIL)
    dstt = dst[TBASE:].reshape(16, TAIL)
    node_b2 = node_b[_P256].reshape(2, 1, HH)
    batch2d = batch.reshape(5, 1, N // 5)
    oW = out_W[_P256].reshape(2, HH, 1)
    ob11 = out_b.reshape(1, 1)

    h = _node_encode(x, node_W[:, _P256], node_b2)

    layer_params = [
        (edge_W0, edge_b0, W1_0, b1_0, W2_0, b2_0),
        (edge_W1, edge_b1, W1_1, b1_1, W2_1, b2_1),
        (edge_W2, edge_b2, W1_2, b1_2, W2_2, b2_2),
    ]
    for (eW, eb, W1, b1, W2, b2) in layer_params:
        e = _edge_proj(edge_attr, eW[:, _P256], eb[_P256].reshape(2, 1, HH))
        h32 = jax.lax.bitcast_convert_type(h.reshape(2 * N, HH // 2, 2), F32)
        e32 = jax.lax.bitcast_convert_type(e.reshape(2 * E, HH // 2, 2), F32)
        aggr = _sc_message(h32, e32, srcm, dstm, srct, dstt)
        h = _mlp(h, aggr, W1[_P256, :], W1, b1.reshape(1, H),
                 W2[:, _P256], b2[_P256].reshape(2, 1, HH))

    return _pool_project(h, batch2d, oW[0], oW[1], ob11)


# speedup vs baseline: 4.3387x; 4.3387x over previous
import functools

import jax
import jax.numpy as jnp
from jax import lax
from jax.experimental import pallas as pl
from jax.experimental.pallas import tpu as pltpu
from jax.experimental.pallas import tpu_sc as plsc

N = 10000
E = 320000
D_IN = 128
D_E = 16
H = 256
HH = 128
G = 64

F32 = jnp.float32


def _node_enc_body(xb, wb, bb, ob):
    ob[...] = (
        jnp.dot(xb[...], wb[...], preferred_element_type=F32)
        + bb[...].reshape(1, HH)
    )


def _node_encode(x, node_W, node_b2):
    nt = 10
    bm = N // nt
    return pl.pallas_call(
        _node_enc_body,
        grid=(2, nt),
        in_specs=[
            pl.BlockSpec((bm, D_IN), lambda c, i: (i, 0)),
            pl.BlockSpec((D_IN, HH), lambda c, i: (0, c)),
            pl.BlockSpec((1, 1, HH), lambda c, i: (c, 0, 0)),
        ],
        out_specs=pl.BlockSpec((bm, HH), lambda c, i: (c * nt + i, 0)),
        out_shape=jax.ShapeDtypeStruct((2 * N, HH), F32),
    )(x, node_W, node_b2)


def _edge_proj_body(ab, wb, bb, ob):
    ob[...] = (
        jnp.dot(ab[...], wb[...], preferred_element_type=F32)
        + bb[...].reshape(1, HH)
    )


def _edge_proj(edge_attr, eW, eb2):
    nt = 80
    bm = E // nt
    return pl.pallas_call(
        _edge_proj_body,
        grid=(2, nt),
        in_specs=[
            pl.BlockSpec((bm, D_E), lambda c, i: (i, 0)),
            pl.BlockSpec((D_E, HH), lambda c, i: (0, c)),
            pl.BlockSpec((1, 1, HH), lambda c, i: (c, 0, 0)),
        ],
        out_specs=pl.BlockSpec((bm, HH), lambda c, i: (c * nt + i, 0)),
        out_shape=jax.ShapeDtypeStruct((2 * E, HH), F32),
    )(edge_attr, eW, eb2)


def _mlp_body(h0, h1, a0, a1, w1, b1, w2, b2, ob):
    z = jnp.concatenate([h0[...] + a0[...], h1[...] + a1[...]], axis=1)
    t = jnp.maximum(jnp.dot(z, w1[...], preferred_element_type=F32) + b1[...], 0.0)
    u = jnp.dot(t, w2[...], preferred_element_type=F32) + b2[...].reshape(1, HH)
    ob[...] = jnp.maximum(u, 0.0)


def _mlp(h_flat, aggr, W1, b1r, W2, b2r):
    nt = 10
    bm = N // nt
    return pl.pallas_call(
        _mlp_body,
        grid=(2, nt),
        in_specs=[
            pl.BlockSpec((bm, HH), lambda c, i: (i, 0)),
            pl.BlockSpec((bm, HH), lambda c, i: (nt + i, 0)),
            pl.BlockSpec((bm, HH), lambda c, i: (i, 0)),
            pl.BlockSpec((bm, HH), lambda c, i: (nt + i, 0)),
            pl.BlockSpec((H, H), lambda c, i: (0, 0)),
            pl.BlockSpec((1, H), lambda c, i: (0, 0)),
            pl.BlockSpec((H, HH), lambda c, i: (0, c)),
            pl.BlockSpec((1, 1, HH), lambda c, i: (c, 0, 0)),
        ],
        out_specs=pl.BlockSpec((bm, HH), lambda c, i: (c * nt + i, 0)),
        out_shape=jax.ShapeDtypeStruct((2 * N, HH), F32),
    )(h_flat, h_flat, aggr, aggr, W1, b1r, W2, b2r)


def _pool_body(h0, h1, bt, w0, w1, bb, ob):
    i = pl.program_id(0)
    y = (
        jnp.dot(h0[...], w0[...], preferred_element_type=F32)
        + jnp.dot(h1[...], w1[...], preferred_element_type=F32)
    )
    bm = y.shape[0]
    oh = (
        lax.broadcasted_iota(jnp.int32, (G, bm), 0)
        == jnp.broadcast_to(bt[...].reshape(1, bm), (G, bm))
    ).astype(F32)
    part = jnp.dot(oh, y, preferred_element_type=F32)

    @pl.when(i == 0)
    def _init():
        ob[...] = jnp.broadcast_to(bb[...], ob.shape)

    ob[...] += part


def _pool_project(h_flat, batch2d, oW0, oW1, ob11):
    nt = 10
    bm = N // nt
    res = pl.pallas_call(
        _pool_body,
        grid=(nt,),
        in_specs=[
            pl.BlockSpec((bm, HH), lambda i: (i, 0)),
            pl.BlockSpec((bm, HH), lambda i: (nt + i, 0)),
            pl.BlockSpec((1, 1, bm), lambda i: (i, 0, 0)),
            pl.BlockSpec((HH, 1), lambda i: (0, 0)),
            pl.BlockSpec((HH, 1), lambda i: (0, 0)),
            pl.BlockSpec((1, 1), lambda i: (0, 0)),
        ],
        out_specs=pl.BlockSpec((G, 1), lambda i: (0, 0)),
        out_shape=jax.ShapeDtypeStruct((G, 1), F32),
    )(h_flat, h_flat, batch2d, oW0, oW1, ob11)
    return res.reshape(G)


CHM = 128
NCM = 156
EPS = NCM * CHM
TAIL = 32
TBASE = 16 * EPS
RPT = 624


def _sc_msg_body(h_hbm, e_hbm, src3_hbm, dst3_hbm, srct_hbm, dstt_hbm, out_hbm,
                 sI0, sI1, dI0, dI1, stx, dtx, buf0, buf1, acc,
                 semE0, semE1, semS0, semS1, semD0, semD1, semG0, semG1):
    c = lax.axis_index("c")
    s = lax.axis_index("s")
    cN = c * N
    cE = c * E
    sI = (sI0, sI1)
    dI = (dI0, dI1)
    buf = (buf0, buf1)
    semE = (semE0, semE1)
    semS = (semS0, semS1)
    semD = (semD0, semD1)
    semG = (semG0, semG1)
    MAXROW = 16 * NCM - 1

    pltpu.sync_copy(srct_hbm.at[s], stx)
    pltpu.sync_copy(dstt_hbm.at[s], dtx)
    for kk in range(TAIL // 16):
        stx[pl.ds(kk * 16, 16)] = stx[pl.ds(kk * 16, 16)] + cN

    zero16 = jnp.zeros((16,), F32)

    def _zrow(r, carry):
        for kk in range(8):
            buf0[r, pl.ds(kk * 16, 16)] = zero16
        return carry

    lax.fori_loop(0, CHM, _zrow, 0)

    arow = s * RPT
    for t in range(RPT // CHM):
        pltpu.sync_copy(buf0, acc.at[pl.ds(arow + t * CHM, CHM)])
    rem = RPT - (RPT // CHM) * CHM
    if rem:
        pltpu.sync_copy(buf0.at[pl.ds(0, rem)],
                        acc.at[pl.ds(arow + (RPT // CHM) * CHM, rem)])

    @pl.when(s == 15)
    def _ztail():
        pltpu.sync_copy(buf0.at[pl.ds(0, N - 16 * RPT)],
                        acc.at[pl.ds(16 * RPT, N - 16 * RPT)])

    plsc.subcore_barrier()

    ebase = cE + s * EPS

    def _prefetch(j, p):
        row = jnp.minimum(s * NCM + j, MAXROW)
        pltpu.async_copy(e_hbm.at[pl.ds(ebase + j * CHM, CHM)], buf[p], semE[p])
        pltpu.async_copy(src3_hbm.at[row], sI[p], semS[p])
        pltpu.async_copy(dst3_hbm.at[row], dI[p], semD[p])

    def _start_gather(p):
        pltpu.make_async_copy(e_hbm.at[pl.ds(0, CHM)], buf[p], semE[p]).wait()
        pltpu.make_async_copy(src3_hbm.at[0], sI[p], semS[p]).wait()
        for kk in range(8):
            sI[p][pl.ds(kk * 16, 16)] = sI[p][pl.ds(kk * 16, 16)] + cN
        pltpu.async_copy(h_hbm.at[sI[p]], buf[p], semG[p], add=True)

    def _relu(b, nrow):
        def _rrow(r, cr):
            for kk in range(8):
                v = b[r, pl.ds(kk * 16, 16)]
                b[r, pl.ds(kk * 16, 16)] = jnp.maximum(v, 0.0)
            return cr
        lax.fori_loop(0, nrow, _rrow, 0)

    def _finish(p):
        pltpu.make_async_copy(h_hbm.at[sI[p]], buf[p], semG[p]).wait()
        _relu(buf[p], CHM)
        pltpu.make_async_copy(dst3_hbm.at[0], dI[p], semD[p]).wait()
        pltpu.sync_copy(buf[p], acc.at[dI[p]], add=True)

    _prefetch(0, 0)
    _prefetch(1, 1)
    _start_gather(0)

    def _body(i, carry):
        j0 = 2 * i
        _start_gather(1)
        _finish(0)
        _prefetch(j0 + 2, 0)
        _finish(1)
        _prefetch(j0 + 3, 1)

        @pl.when(i < NCM // 2 - 1)
        def _more():
            _start_gather(0)
        return carry

    lax.fori_loop(0, NCM // 2, _body, 0)

    for p in range(2):
        pltpu.make_async_copy(e_hbm.at[pl.ds(0, CHM)], buf[p], semE[p]).wait()
        pltpu.make_async_copy(src3_hbm.at[0], sI[p], semS[p]).wait()
        pltpu.make_async_copy(dst3_hbm.at[0], dI[p], semD[p]).wait()

    pltpu.sync_copy(e_hbm.at[pl.ds(cE + TBASE + s * TAIL, TAIL)],
                    buf0.at[pl.ds(0, TAIL)])
    pltpu.sync_copy(h_hbm.at[stx], buf0.at[pl.ds(0, TAIL)], add=True)
    _relu(buf0, TAIL)
    pltpu.sync_copy(buf0.at[pl.ds(0, TAIL)], acc.at[dtx], add=True)

    plsc.subcore_barrier()
    pltpu.sync_copy(acc.at[pl.ds(arow, RPT)],
                    out_hbm.at[pl.ds(cN + arow, RPT)])

    @pl.when(s == 15)
    def _wtail():
        pltpu.sync_copy(acc.at[pl.ds(16 * RPT, N - 16 * RPT)],
                        out_hbm.at[pl.ds(cN + 16 * RPT, N - 16 * RPT)])


def _sc_message(h_flat, e_flat, src3, dst3, srct, dstt):
    mesh = plsc.VectorSubcoreMesh(core_axis_name="c", subcore_axis_name="s")
    k = functools.partial(
        pl.kernel,
        out_type=jax.ShapeDtypeStruct((2 * N, HH), F32),
        mesh=mesh,
        scratch_types=[
            pltpu.VMEM((CHM,), jnp.int32),
            pltpu.VMEM((CHM,), jnp.int32),
            pltpu.VMEM((CHM,), jnp.int32),
            pltpu.VMEM((CHM,), jnp.int32),
            pltpu.VMEM((TAIL,), jnp.int32),
            pltpu.VMEM((TAIL,), jnp.int32),
            pltpu.VMEM((CHM, HH), F32),
            pltpu.VMEM((CHM, HH), F32),
            pltpu.VMEM_SHARED((N, HH), F32),
            pltpu.SemaphoreType.DMA,
            pltpu.SemaphoreType.DMA,
            pltpu.SemaphoreType.DMA,
            pltpu.SemaphoreType.DMA,
            pltpu.SemaphoreType.DMA,
            pltpu.SemaphoreType.DMA,
            pltpu.SemaphoreType.DMA,
            pltpu.SemaphoreType.DMA,
        ],
    )(_sc_msg_body)
    return k(h_flat, e_flat, src3, dst3, srct, dstt)


def kernel(x, edge_index, edge_attr, batch, node_W, node_b,
           edge_W0, edge_b0, W1_0, b1_0, W2_0, b2_0,
           edge_W1, edge_b1, W1_1, b1_1, W2_1, b2_1,
           edge_W2, edge_b2, W1_2, b1_2, W2_2, b2_2,
           out_W, out_b):
    src = edge_index[0]
    dst = edge_index[1]
    srcm = src[:TBASE].reshape(16 * NCM, CHM)
    dstm = dst[:TBASE].reshape(16 * NCM, CHM)
    srct = src[TBASE:].reshape(16, TAIL)
    dstt = dst[TBASE:].reshape(16, TAIL)
    node_b2 = node_b.reshape(2, 1, HH)
    batch2d = batch.reshape(10, 1, N // 10)
    oW = out_W.reshape(2, HH, 1)
    ob11 = out_b.reshape(1, 1)

    h = _node_encode(x, node_W, node_b2)

    layer_params = [
        (edge_W0, edge_b0, W1_0, b1_0, W2_0, b2_0),
        (edge_W1, edge_b1, W1_1, b1_1, W2_1, b2_1),
        (edge_W2, edge_b2, W1_2, b1_2, W2_2, b2_2),
    ]
    for (eW, eb, W1, b1, W2, b2) in layer_params:
        e = _edge_proj(edge_attr, eW, eb.reshape(2, 1, HH))
        aggr = _sc_message(h, e, srcm, dstm, srct, dstt)
        h = _mlp(h, aggr, W1, b1.reshape(1, H), W2, b2.reshape(2, 1, HH))

    return _pool_project(h, batch2d, oW[0], oW[1], ob11)
